# Initial kernel scaffold; baseline (speedup 1.0000x reference)
import functools

import jax
import jax.numpy as jnp
from jax import lax
from jax.experimental import pallas as pl
from jax.experimental.pallas import tpu as pltpu
from jax.experimental.pallas import tpu_sc as plsc

N = 100000
E = 1600000
SD = 32
VD = 8

NC = 2
NS = 16
NW = NC * NS

F32 = jnp.float32
I32 = jnp.int32

TS_W = 64
TR_W = 48

CH = 80
EPW = E // NW
NCH_G = EPW // CH
EPT = E // NS
NCH_S = EPT // CH
HALF = N // 2
RPT = HALF // NS

_sc_mesh = plsc.VectorSubcoreMesh(core_axis_name="c", subcore_axis_name="s")


@functools.partial(
    pl.kernel,
    out_type=(jax.ShapeDtypeStruct((E, TS_W), F32),
              jax.ShapeDtypeStruct((E, TR_W), F32)),
    mesh=_sc_mesh,
    scratch_types=[
        pltpu.VMEM((CH,), I32),
        pltpu.VMEM((CH,), I32),
        pltpu.VMEM((CH, TS_W), F32),
        pltpu.VMEM((CH, TR_W), F32),
        pltpu.SemaphoreType.DMA,
        pltpu.SemaphoreType.DMA,
    ],
)
def _gather_edges(ts_hbm, tr_hbm, snd_hbm, rcv_hbm, xs_hbm, xr_hbm,
                  sidx, ridx, srow, rrow, sem_s, sem_r):
    wid = lax.axis_index("s") * NC + lax.axis_index("c")
    base = wid * EPW

    def body(i, carry):
        e0 = base + i * CH
        pltpu.sync_copy(snd_hbm.at[pl.ds(e0, CH)], sidx)
        pltpu.sync_copy(rcv_hbm.at[pl.ds(e0, CH)], ridx)
        cs = pltpu.async_copy(ts_hbm.at[sidx], srow, sem_s)
        cr = pltpu.async_copy(tr_hbm.at[ridx], rrow, sem_r)
        cs.wait()
        cr.wait()
        pltpu.sync_copy(srow, xs_hbm.at[pl.ds(e0, CH)])
        pltpu.sync_copy(rrow, xr_hbm.at[pl.ds(e0, CH)])
        return carry

    lax.fori_loop(0, NCH_G, body, 0)


def _make_scatter(width):
    @functools.partial(
        pl.kernel,
        out_type=jax.ShapeDtypeStruct((N, width), F32),
        mesh=_sc_mesh,
        scratch_types=[
            pltpu.VMEM((CH,), I32),
            pltpu.VMEM((CH,), I32),
            pltpu.VMEM((CH, width), F32),
            pltpu.VMEM_SHARED((HALF + 8, width), F32),
        ],
    )
    def _scatter(msg_hbm, rcv_hbm, base_hbm, out_hbm, idx_raw, idx_loc, rows, acc):
        c = lax.axis_index("c")
        s = lax.axis_index("s")
        nb = c * HALF
        pltpu.sync_copy(base_hbm.at[pl.ds(nb + s * RPT, RPT)],
                        acc.at[pl.ds(s * RPT, RPT)])
        plsc.subcore_barrier()

        def body(i, carry):
            e0 = s * EPT + i * CH
            pltpu.sync_copy(rcv_hbm.at[pl.ds(e0, CH)], idx_raw)
            for k in range(CH // 16):
                idx = idx_raw[pl.ds(k * 16, 16)]
                loc = idx - nb
                inb = (loc >= 0) & (loc < HALF)
                idx_loc[pl.ds(k * 16, 16)] = jnp.where(inb, loc, HALF)
            pltpu.sync_copy(msg_hbm.at[pl.ds(e0, CH)], rows)
            pltpu.sync_copy(rows, acc.at[idx_loc], add=True)
            return carry

        lax.fori_loop(0, NCH_S, body, 0)
        plsc.subcore_barrier()
        pltpu.sync_copy(acc.at[pl.ds(s * RPT, RPT)],
                        out_hbm.at[pl.ds(nb + s * RPT, RPT)])

    return _scatter


_scatter_h = _make_scatter(SD)
_scatter_v = _make_scatter(2 * VD)


def _silu(x):
    return x * (1.0 / (1.0 + jnp.exp(-x)))


BT = 2000
GT = E // BT


def _tc_body(xs_ref, xr_ref, w1_ref, b1_ref, w2_ref, b2_ref, w3_ref, b3_ref,
             mh_ref, mv_ref):
    xs = xs_ref[...]
    xr = xr_ref[...]
    h_s = xs[:, 0:SD]
    vx = xs[:, SD:SD + VD]
    vy = xs[:, SD + VD:SD + 2 * VD]
    pxs = xs[:, 48:49]
    pys = xs[:, 49:50]
    th_i = xs[:, 50:51]
    h_r = xr[:, 0:SD]
    pxr = xr[:, 32:33]
    pyr = xr[:, 33:34]
    th_j = xr[:, 34:35]

    dx = pxs - pxr
    dy = pys - pyr
    rr = jnp.sqrt(dx * dx + dy * dy)
    inv = 1.0 / (rr + 1e-8)
    ux = dx * inv
    uy = dy * inv
    dth = th_i - th_j
    cj = jnp.cos(th_j)
    sj = jnp.sin(th_j)
    cd = jnp.cos(dth)
    sd = jnp.sin(dth)
    urx = ux * cj + uy * sj
    ury = uy * cj - ux * sj
    wx = vx * cd - vy * sd
    wy = vx * sd + vy * cd
    vn = jnp.sqrt(wx * wx + wy * wy)
    vdot = wx * urx + wy * ury

    pad = jnp.zeros((BT, 128 - 83), F32)
    x_in = jnp.concatenate([h_s, h_r, rr, cd, sd, vn, vdot, pad], axis=1)
    h1 = _silu(jnp.dot(x_in, w1_ref[...], preferred_element_type=F32) + b1_ref[...])
    h2 = _silu(jnp.dot(h1, w2_ref[...], preferred_element_type=F32) + b2_ref[...])
    o = jnp.dot(h2, w3_ref[...], preferred_element_type=F32) + b3_ref[...]

    a = o[:, 0:8]
    b = o[:, 8:16]
    cgate = o[:, 16:24]
    dh = o[:, 24:56]
    psi = o[:, 56:64]
    g = b + cgate * psi
    mh_ref[...] = dh
    mv_ref[...] = jnp.concatenate([a * wx + g * urx, a * wy + g * ury], axis=1)


_tc_call = pl.pallas_call(
    _tc_body,
    grid=(GT,),
    in_specs=[
        pl.BlockSpec((BT, TS_W), lambda i: (i, 0)),
        pl.BlockSpec((BT, TR_W), lambda i: (i, 0)),
        pl.BlockSpec((128, 192), lambda i: (0, 0)),
        pl.BlockSpec((1, 192), lambda i: (0, 0)),
        pl.BlockSpec((192, 192), lambda i: (0, 0)),
        pl.BlockSpec((1, 192), lambda i: (0, 0)),
        pl.BlockSpec((192, 64), lambda i: (0, 0)),
        pl.BlockSpec((1, 64), lambda i: (0, 0)),
    ],
    out_specs=[
        pl.BlockSpec((BT, SD), lambda i: (i, 0)),
        pl.BlockSpec((BT, 2 * VD), lambda i: (i, 0)),
    ],
    out_shape=[
        jax.ShapeDtypeStruct((E, SD), F32),
        jax.ShapeDtypeStruct((E, 2 * VD), F32),
    ],
)


def _pack_weights(gW1, gb1, gW2, gb2, gW3, gb3, pW1, pb1, pW2, pb2, pW3, pb3,
                  sW1, sb1, sW2, sb2, sW3, sb3):
    f = lambda x: x.astype(F32)
    w1c = jnp.zeros((128, 192), F32)
    w1c = w1c.at[0:67, 0:64].set(f(gW1))
    w1c = w1c.at[0:83, 64:128].set(f(sW1))
    w1c = w1c.at[0:32, 128:192].set(f(pW1))
    b1c = jnp.concatenate([f(gb1), f(sb1), f(pb1)]).reshape(1, 192)
    w2c = jnp.zeros((192, 192), F32)
    w2c = w2c.at[0:64, 0:64].set(f(gW2))
    w2c = w2c.at[64:128, 64:128].set(f(sW2))
    w2c = w2c.at[128:192, 128:192].set(f(pW2))
    b2c = jnp.concatenate([f(gb2), f(sb2), f(pb2)]).reshape(1, 192)
    w3c = jnp.zeros((192, 64), F32)
    w3c = w3c.at[0:64, 0:24].set(f(gW3))
    w3c = w3c.at[64:128, 24:56].set(f(sW3))
    w3c = w3c.at[128:192, 56:64].set(f(pW3))
    b3c = jnp.concatenate([f(gb3), f(sb3), f(pb3)]).reshape(1, 64)
    return w1c, b1c, w2c, b2c, w3c, b3c


def kernel(h, v, midpoint_pos, midpoint_theta, senders, receivers,
           gW1, gb1, gW2, gb2, gW3, gb3,
           pW1, pb1, pW2, pb2, pW3, pb3,
           sW1, sb1, sW2, sb2, sW3, sb3):
    h = h.astype(F32)
    v = v.astype(F32)
    pos = midpoint_pos.astype(F32)
    th = midpoint_theta.astype(F32)
    snd = senders.astype(I32)
    rcv = receivers.astype(I32)

    vx = v[:, :, 0]
    vy = v[:, :, 1]
    pad13 = jnp.zeros((N, 13), F32)
    ts = jnp.concatenate([h, vx, vy, pos, th[:, None], pad13], axis=1)
    tr = jnp.concatenate([h, pos, th[:, None], pad13], axis=1)

    xs, xr = _gather_edges(ts, tr, snd, rcv)

    packed = _pack_weights(gW1, gb1, gW2, gb2, gW3, gb3,
                           pW1, pb1, pW2, pb2, pW3, pb3,
                           sW1, sb1, sW2, sb2, sW3, sb3)
    msg_h, msg_v = _tc_call(xs, xr, *packed)

    h_new = _scatter_h(msg_h, rcv, h)
    vflat = jnp.concatenate([vx, vy], axis=1)
    vnew_flat = _scatter_v(msg_v, rcv, vflat)
    v_new = jnp.stack([vnew_flat[:, :VD], vnew_flat[:, VD:]], axis=-1)
    return h_new, v_new


# speedup vs baseline: 12.7108x; 12.7108x over previous
import functools

import jax
import jax.numpy as jnp
from jax import lax
from jax.experimental import pallas as pl
from jax.experimental.pallas import tpu as pltpu
from jax.experimental.pallas import tpu_sc as plsc

N = 100000
E = 1600000
SD = 32
VD = 8

NC = 2
NS = 16
NW = NC * NS

F32 = jnp.float32
I32 = jnp.int32

TS_W = 64
TR_W = 48

CH = 80
EPW = E // NW
NCH_G = EPW // CH
EPT = E // NS
NCH_S = EPT // CH
HALF = N // 2
RPT = HALF // NS

@functools.cache
def _get_gather_edges():
    mesh = plsc.VectorSubcoreMesh(core_axis_name="c", subcore_axis_name="s")

    @functools.partial(
        pl.kernel,
        out_type=(jax.ShapeDtypeStruct((E, TS_W), F32),
                  jax.ShapeDtypeStruct((E, TR_W), F32)),
        mesh=mesh,
        scratch_types=[
            pltpu.VMEM((CH,), I32),
            pltpu.VMEM((CH,), I32),
            pltpu.VMEM((CH, TS_W), F32),
            pltpu.VMEM((CH, TR_W), F32),
            pltpu.SemaphoreType.DMA,
            pltpu.SemaphoreType.DMA,
        ],
        compiler_params=pltpu.CompilerParams(use_tc_tiling_on_sc=False),
    )
    def _gather_edges(ts_hbm, tr_hbm, snd_hbm, rcv_hbm, xs_hbm, xr_hbm,
                      sidx, ridx, srow, rrow, sem_s, sem_r):
        wid = lax.axis_index("s") * NC + lax.axis_index("c")
        base = wid * EPW

        def body(i, carry):
            e0 = base + i * CH
            pltpu.sync_copy(snd_hbm.at[pl.ds(e0, CH)], sidx)
            pltpu.sync_copy(rcv_hbm.at[pl.ds(e0, CH)], ridx)
            cs = pltpu.async_copy(ts_hbm.at[sidx], srow, sem_s)
            cr = pltpu.async_copy(tr_hbm.at[ridx], rrow, sem_r)
            cs.wait()
            cr.wait()
            pltpu.sync_copy(srow, xs_hbm.at[pl.ds(e0, CH)])
            pltpu.sync_copy(rrow, xr_hbm.at[pl.ds(e0, CH)])
            return carry

        lax.fori_loop(0, NCH_G, body, 0)

    return _gather_edges


@functools.cache
def _make_scatter(width):
    mesh = plsc.VectorSubcoreMesh(core_axis_name="c", subcore_axis_name="s")

    @functools.partial(
        pl.kernel,
        out_type=jax.ShapeDtypeStruct((N, width), F32),
        mesh=mesh,
        scratch_types=[
            pltpu.VMEM((CH,), I32),
            pltpu.VMEM((CH,), I32),
            pltpu.VMEM((CH, width), F32),
            pltpu.VMEM_SHARED((HALF + 8, width), F32),
        ],
        compiler_params=pltpu.CompilerParams(use_tc_tiling_on_sc=False),
    )
    def _scatter(msg_hbm, rcv_hbm, base_hbm, out_hbm, idx_raw, idx_loc, rows, acc):
        c = lax.axis_index("c")
        s = lax.axis_index("s")
        nb = c * HALF
        pltpu.sync_copy(base_hbm.at[pl.ds(nb + s * RPT, RPT)],
                        acc.at[pl.ds(s * RPT, RPT)])
        plsc.subcore_barrier()

        def body(i, carry):
            e0 = s * EPT + i * CH
            pltpu.sync_copy(rcv_hbm.at[pl.ds(e0, CH)], idx_raw)
            for k in range(CH // 16):
                idx = idx_raw[pl.ds(k * 16, 16)]
                loc = idx - nb
                inb = (loc >= 0) & (loc < HALF)
                idx_loc[pl.ds(k * 16, 16)] = jnp.where(inb, loc, HALF)
            pltpu.sync_copy(msg_hbm.at[pl.ds(e0, CH)], rows)
            pltpu.sync_copy(rows, acc.at[idx_loc], add=True)
            return carry

        lax.fori_loop(0, NCH_S, body, 0)
        plsc.subcore_barrier()
        pltpu.sync_copy(acc.at[pl.ds(s * RPT, RPT)],
                        out_hbm.at[pl.ds(nb + s * RPT, RPT)])

    return _scatter


def _silu(x):
    return x * (1.0 / (1.0 + jnp.exp(-x)))


BT = 2000
GT = E // BT


def _edge_compute(xs, xr, w1, b1, w2, b2, w3, b3):
    nrows = xs.shape[0]
    h_s = xs[:, 0:SD]
    vx = xs[:, SD:SD + VD]
    vy = xs[:, SD + VD:SD + 2 * VD]
    pxs = xs[:, 48:49]
    pys = xs[:, 49:50]
    th_i = xs[:, 50:51]
    h_r = xr[:, 0:SD]
    pxr = xr[:, 32:33]
    pyr = xr[:, 33:34]
    th_j = xr[:, 34:35]

    dx = pxs - pxr
    dy = pys - pyr
    rr = jnp.sqrt(dx * dx + dy * dy)
    inv = 1.0 / (rr + 1e-8)
    ux = dx * inv
    uy = dy * inv
    dth = th_i - th_j
    cj = jnp.cos(th_j)
    sj = jnp.sin(th_j)
    cd = jnp.cos(dth)
    sd = jnp.sin(dth)
    urx = ux * cj + uy * sj
    ury = uy * cj - ux * sj
    wx = vx * cd - vy * sd
    wy = vx * sd + vy * cd
    vn = jnp.sqrt(wx * wx + wy * wy)
    vdot = wx * urx + wy * ury

    pad = jnp.zeros((nrows, 128 - 83), F32)
    x_in = jnp.concatenate([h_s, h_r, rr, cd, sd, vn, vdot, pad], axis=1)
    h1 = _silu(jnp.dot(x_in, w1, preferred_element_type=F32) + b1)
    h2 = _silu(jnp.dot(h1, w2, preferred_element_type=F32) + b2)
    o = jnp.dot(h2, w3, preferred_element_type=F32) + b3

    a = o[:, 0:8]
    b = o[:, 8:16]
    cgate = o[:, 16:24]
    dh = o[:, 24:56]
    psi = o[:, 56:64]
    g = b + cgate * psi
    mv = jnp.concatenate([a * wx + g * urx, a * wy + g * ury], axis=1)
    return dh, mv


def _tc_body(xs_ref, xr_ref, w1_ref, b1_ref, w2_ref, b2_ref, w3_ref, b3_ref,
             mh_ref, mv_ref):
    mh, mv = _edge_compute(xs_ref[...], xr_ref[...], w1_ref[...], b1_ref[...],
                           w2_ref[...], b2_ref[...], w3_ref[...], b3_ref[...])
    mh_ref[...] = mh
    mv_ref[...] = mv


_tc_call = pl.pallas_call(
    _tc_body,
    grid=(GT,),
    in_specs=[
        pl.BlockSpec((BT, TS_W), lambda i: (i, 0)),
        pl.BlockSpec((BT, TR_W), lambda i: (i, 0)),
        pl.BlockSpec((128, 192), lambda i: (0, 0)),
        pl.BlockSpec((1, 192), lambda i: (0, 0)),
        pl.BlockSpec((192, 192), lambda i: (0, 0)),
        pl.BlockSpec((1, 192), lambda i: (0, 0)),
        pl.BlockSpec((192, 64), lambda i: (0, 0)),
        pl.BlockSpec((1, 64), lambda i: (0, 0)),
    ],
    out_specs=[
        pl.BlockSpec((BT, SD), lambda i: (i, 0)),
        pl.BlockSpec((BT, 2 * VD), lambda i: (i, 0)),
    ],
    out_shape=[
        jax.ShapeDtypeStruct((E, SD), F32),
        jax.ShapeDtypeStruct((E, 2 * VD), F32),
    ],
)


def _pack_weights(gW1, gb1, gW2, gb2, gW3, gb3, pW1, pb1, pW2, pb2, pW3, pb3,
                  sW1, sb1, sW2, sb2, sW3, sb3):
    f = lambda x: x.astype(F32)
    w1c = jnp.zeros((128, 192), F32)
    w1c = w1c.at[0:67, 0:64].set(f(gW1))
    w1c = w1c.at[0:83, 64:128].set(f(sW1))
    w1c = w1c.at[0:32, 128:192].set(f(pW1))
    b1c = jnp.concatenate([f(gb1), f(sb1), f(pb1)]).reshape(1, 192)
    w2c = jnp.zeros((192, 192), F32)
    w2c = w2c.at[0:64, 0:64].set(f(gW2))
    w2c = w2c.at[64:128, 64:128].set(f(sW2))
    w2c = w2c.at[128:192, 128:192].set(f(pW2))
    b2c = jnp.concatenate([f(gb2), f(sb2), f(pb2)]).reshape(1, 192)
    w3c = jnp.zeros((192, 64), F32)
    w3c = w3c.at[0:64, 0:24].set(f(gW3))
    w3c = w3c.at[64:128, 24:56].set(f(sW3))
    w3c = w3c.at[128:192, 56:64].set(f(pW3))
    b3c = jnp.concatenate([f(gb3), f(sb3), f(pb3)]).reshape(1, 64)
    return w1c, b1c, w2c, b2c, w3c, b3c


def kernel(h, v, midpoint_pos, midpoint_theta, senders, receivers,
           gW1, gb1, gW2, gb2, gW3, gb3,
           pW1, pb1, pW2, pb2, pW3, pb3,
           sW1, sb1, sW2, sb2, sW3, sb3):
    h = h.astype(F32)
    v = v.astype(F32)
    pos = midpoint_pos.astype(F32)
    th = midpoint_theta.astype(F32)
    snd = senders.astype(I32)
    rcv = receivers.astype(I32)

    vx = v[:, :, 0]
    vy = v[:, :, 1]
    pad13 = jnp.zeros((N, 13), F32)
    ts = jnp.concatenate([h, vx, vy, pos, th[:, None], pad13], axis=1)
    tr = jnp.concatenate([h, pos, th[:, None], pad13], axis=1)

    xs, xr = _get_gather_edges()(ts, tr, snd, rcv)

    packed = _pack_weights(gW1, gb1, gW2, gb2, gW3, gb3,
                           pW1, pb1, pW2, pb2, pW3, pb3,
                           sW1, sb1, sW2, sb2, sW3, sb3)
    msg_h, msg_v = _tc_call(xs, xr, *packed)

    h_new = _make_scatter(SD)(msg_h, rcv, h)
    vflat = jnp.concatenate([vx, vy], axis=1)
    vnew_flat = _make_scatter(2 * VD)(msg_v, rcv, vflat)
    v_new = jnp.stack([vnew_flat[:, :VD], vnew_flat[:, VD:]], axis=-1)
    return h_new, v_new


# speedup vs baseline: 17.2459x; 1.3568x over previous
import functools

import jax
import jax.numpy as jnp
from jax import lax
from jax.experimental import pallas as pl
from jax.experimental.pallas import tpu as pltpu
from jax.experimental.pallas import tpu_sc as plsc

N = 100000
E = 1600000
SD = 32
VD = 8

NC = 2
NS = 16
NW = NC * NS

F32 = jnp.float32
I32 = jnp.int32

TS_W = 64
TR_W = 48

CH = 80
EPW = E // NW
NCH_G = EPW // CH
EPT = E // NS
NCH_S = EPT // CH
HALF = N // 2
RPT = HALF // NS

@functools.cache
def _get_gather_edges():
    mesh = plsc.VectorSubcoreMesh(core_axis_name="c", subcore_axis_name="s")

    @functools.partial(
        pl.kernel,
        out_type=(jax.ShapeDtypeStruct((E, TS_W), F32),
                  jax.ShapeDtypeStruct((E, TR_W), F32)),
        mesh=mesh,
        scratch_types=[
            pltpu.VMEM((CH,), I32),
            pltpu.VMEM((CH,), I32),
            pltpu.VMEM((CH, TS_W), F32),
            pltpu.VMEM((CH, TR_W), F32),
            pltpu.SemaphoreType.DMA,
            pltpu.SemaphoreType.DMA,
        ],
        compiler_params=pltpu.CompilerParams(use_tc_tiling_on_sc=False),
    )
    def _gather_edges(ts_hbm, tr_hbm, snd_hbm, rcv_hbm, xs_hbm, xr_hbm,
                      sidx, ridx, srow, rrow, sem_s, sem_r):
        wid = lax.axis_index("s") * NC + lax.axis_index("c")
        base = wid * EPW

        def body(i, carry):
            e0 = base + i * CH
            pltpu.sync_copy(snd_hbm.at[pl.ds(e0, CH)], sidx)
            pltpu.sync_copy(rcv_hbm.at[pl.ds(e0, CH)], ridx)
            cs = pltpu.async_copy(ts_hbm.at[sidx], srow, sem_s)
            cr = pltpu.async_copy(tr_hbm.at[ridx], rrow, sem_r)
            cs.wait()
            cr.wait()
            pltpu.sync_copy(srow, xs_hbm.at[pl.ds(e0, CH)])
            pltpu.sync_copy(rrow, xr_hbm.at[pl.ds(e0, CH)])
            return carry

        lax.fori_loop(0, NCH_G, body, 0)

    return _gather_edges


@functools.cache
def _make_scatter(width):
    mesh = plsc.VectorSubcoreMesh(core_axis_name="c", subcore_axis_name="s")

    @functools.partial(
        pl.kernel,
        out_type=jax.ShapeDtypeStruct((N, width), F32),
        mesh=mesh,
        scratch_types=[
            pltpu.VMEM((CH,), I32),
            pltpu.VMEM((CH,), I32),
            pltpu.VMEM((CH, width), F32),
            pltpu.VMEM_SHARED((HALF + 8, width), F32),
        ],
        compiler_params=pltpu.CompilerParams(use_tc_tiling_on_sc=False),
    )
    def _scatter(msg_hbm, rcv_hbm, base_hbm, out_hbm, idx_raw, idx_loc, rows, acc):
        c = lax.axis_index("c")
        s = lax.axis_index("s")
        nb = c * HALF
        pltpu.sync_copy(base_hbm.at[pl.ds(nb + s * RPT, RPT)],
                        acc.at[pl.ds(s * RPT, RPT)])
        plsc.subcore_barrier()

        def body(i, carry):
            e0 = s * EPT + i * CH
            pltpu.sync_copy(rcv_hbm.at[pl.ds(e0, CH)], idx_raw)
            for k in range(CH // 16):
                idx = idx_raw[pl.ds(k * 16, 16)]
                loc = idx - nb
                inb = (loc >= 0) & (loc < HALF)
                idx_loc[pl.ds(k * 16, 16)] = jnp.where(inb, loc, HALF)
            pltpu.sync_copy(msg_hbm.at[pl.ds(e0, CH)], rows)
            pltpu.sync_copy(rows, acc.at[idx_loc], add=True)
            return carry

        lax.fori_loop(0, NCH_S, body, 0)
        plsc.subcore_barrier()
        pltpu.sync_copy(acc.at[pl.ds(s * RPT, RPT)],
                        out_hbm.at[pl.ds(nb + s * RPT, RPT)])

    return _scatter


def _silu(x):
    return x * (1.0 / (1.0 + jnp.exp(-x)))


def _prologue_body(vxt_ref, vyt_ref, tht_ref, vrxt_ref, vryt_ref, nrmt_ref,
                   ct_ref, st_ref):
    th = tht_ref[...]
    c = jnp.cos(th)
    s = jnp.sin(th)
    ct_ref[...] = c
    st_ref[...] = s
    vx = vxt_ref[...]
    vy = vyt_ref[...]
    vrxt_ref[...] = vx * c - vy * s
    vryt_ref[...] = vx * s + vy * c
    nrmt_ref[...] = jnp.sqrt(vx * vx + vy * vy)


_prologue = pl.pallas_call(
    _prologue_body,
    out_shape=[
        jax.ShapeDtypeStruct((VD, N), F32),
        jax.ShapeDtypeStruct((VD, N), F32),
        jax.ShapeDtypeStruct((VD, N), F32),
        jax.ShapeDtypeStruct((1, N), F32),
        jax.ShapeDtypeStruct((1, N), F32),
    ],
)


def _epilogue_body(acc_ref, ct_ref, st_ref, out_ref):
    ax = acc_ref[0:VD, :]
    ay = acc_ref[VD:2 * VD, :]
    c = ct_ref[...]
    s = st_ref[...]
    nx = ax * c + ay * s
    ny = ay * c - ax * s
    out_ref[0:VD, :] = nx
    out_ref[VD:2 * VD, :] = ny


_epilogue = pl.pallas_call(
    _epilogue_body,
    out_shape=jax.ShapeDtypeStruct((2 * VD, N), F32),
)


BT = 2000
GT = E // BT


def _edge_compute(xs, xr, w1, b1, w2, b2, w3, b3):
    nrows = xs.shape[0]
    h_s = xs[:, 0:SD]
    vrx = xs[:, SD:SD + VD]
    vry = xs[:, SD + VD:SD + 2 * VD]
    nrm = xs[:, 48:56]
    pxs = xs[:, 56:57]
    pys = xs[:, 57:58]
    ci = xs[:, 58:59]
    si = xs[:, 59:60]
    h_r = xr[:, 0:SD]
    pxr = xr[:, 32:33]
    pyr = xr[:, 33:34]
    cj = xr[:, 34:35]
    sj = xr[:, 35:36]

    dx = pxs - pxr
    dy = pys - pyr
    rr = jnp.sqrt(dx * dx + dy * dy)
    inv = 1.0 / (rr + 1e-8)
    ux = dx * inv
    uy = dy * inv
    cd = ci * cj + si * sj
    sd = si * cj - ci * sj
    vdot = vrx * ux + vry * uy

    pad = jnp.zeros((nrows, 128 - 83), F32)
    x_in = jnp.concatenate([h_s, h_r, rr, cd, sd, nrm, vdot, pad], axis=1)
    h1 = _silu(jnp.dot(x_in, w1, preferred_element_type=F32) + b1)
    h2 = _silu(jnp.dot(h1, w2, preferred_element_type=F32) + b2)
    o = jnp.dot(h2, w3, preferred_element_type=F32) + b3

    a = o[:, 0:8]
    b = o[:, 8:16]
    cgate = o[:, 16:24]
    dh = o[:, 24:56]
    psi = o[:, 56:64]
    g = b + cgate * psi
    mv = jnp.concatenate([a * vrx + g * ux, a * vry + g * uy], axis=1)
    return dh, mv


def _tc_body(xs_ref, xr_ref, w1_ref, b1_ref, w2_ref, b2_ref, w3_ref, b3_ref,
             mh_ref, mv_ref):
    mh, mv = _edge_compute(xs_ref[...], xr_ref[...], w1_ref[...], b1_ref[...],
                           w2_ref[...], b2_ref[...], w3_ref[...], b3_ref[...])
    mh_ref[...] = mh
    mv_ref[...] = mv


_tc_call = pl.pallas_call(
    _tc_body,
    grid=(GT,),
    in_specs=[
        pl.BlockSpec((BT, TS_W), lambda i: (i, 0)),
        pl.BlockSpec((BT, TR_W), lambda i: (i, 0)),
        pl.BlockSpec((128, 192), lambda i: (0, 0)),
        pl.BlockSpec((1, 192), lambda i: (0, 0)),
        pl.BlockSpec((192, 192), lambda i: (0, 0)),
        pl.BlockSpec((1, 192), lambda i: (0, 0)),
        pl.BlockSpec((192, 64), lambda i: (0, 0)),
        pl.BlockSpec((1, 64), lambda i: (0, 0)),
    ],
    out_specs=[
        pl.BlockSpec((BT, SD), lambda i: (i, 0)),
        pl.BlockSpec((BT, 2 * VD), lambda i: (i, 0)),
    ],
    out_shape=[
        jax.ShapeDtypeStruct((E, SD), F32),
        jax.ShapeDtypeStruct((E, 2 * VD), F32),
    ],
)


def _pack_weights(gW1, gb1, gW2, gb2, gW3, gb3, pW1, pb1, pW2, pb2, pW3, pb3,
                  sW1, sb1, sW2, sb2, sW3, sb3):
    f = lambda x: x.astype(F32)
    w1c = jnp.zeros((128, 192), F32)
    w1c = w1c.at[0:67, 0:64].set(f(gW1))
    w1c = w1c.at[0:83, 64:128].set(f(sW1))
    w1c = w1c.at[0:32, 128:192].set(f(pW1))
    b1c = jnp.concatenate([f(gb1), f(sb1), f(pb1)]).reshape(1, 192)
    w2c = jnp.zeros((192, 192), F32)
    w2c = w2c.at[0:64, 0:64].set(f(gW2))
    w2c = w2c.at[64:128, 64:128].set(f(sW2))
    w2c = w2c.at[128:192, 128:192].set(f(pW2))
    b2c = jnp.concatenate([f(gb2), f(sb2), f(pb2)]).reshape(1, 192)
    w3c = jnp.zeros((192, 64), F32)
    w3c = w3c.at[0:64, 0:24].set(f(gW3))
    w3c = w3c.at[64:128, 24:56].set(f(sW3))
    w3c = w3c.at[128:192, 56:64].set(f(pW3))
    b3c = jnp.concatenate([f(gb3), f(sb3), f(pb3)]).reshape(1, 64)
    return w1c, b1c, w2c, b2c, w3c, b3c


def kernel(h, v, midpoint_pos, midpoint_theta, senders, receivers,
           gW1, gb1, gW2, gb2, gW3, gb3,
           pW1, pb1, pW2, pb2, pW3, pb3,
           sW1, sb1, sW2, sb2, sW3, sb3):
    h = h.astype(F32)
    v = v.astype(F32)
    pos = midpoint_pos.astype(F32)
    th = midpoint_theta.astype(F32)
    snd = senders.astype(I32)
    rcv = receivers.astype(I32)

    vxt = v[:, :, 0].T
    vyt = v[:, :, 1].T
    vrxt, vryt, nrmt, ct, st = _prologue(vxt, vyt, th.reshape(1, N))
    vrot_x = vrxt.T
    vrot_y = vryt.T
    nrm = nrmt.T
    c_col = ct.reshape(N, 1)
    s_col = st.reshape(N, 1)

    pad4 = jnp.zeros((N, 4), F32)
    pad12 = jnp.zeros((N, 12), F32)
    ts = jnp.concatenate([h, vrot_x, vrot_y, nrm, pos, c_col, s_col, pad4],
                         axis=1)
    tr = jnp.concatenate([h, pos, c_col, s_col, pad12], axis=1)

    xs, xr = _get_gather_edges()(ts, tr, snd, rcv)

    packed = _pack_weights(gW1, gb1, gW2, gb2, gW3, gb3,
                           pW1, pb1, pW2, pb2, pW3, pb3,
                           sW1, sb1, sW2, sb2, sW3, sb3)
    msg_h, msg_v = _tc_call(xs, xr, *packed)

    h_new = _make_scatter(SD)(msg_h, rcv, h)
    vrot_flat = jnp.concatenate([vrot_x, vrot_y], axis=1)
    vacc = _make_scatter(2 * VD)(msg_v, rcv, vrot_flat)
    vnt = _epilogue(vacc.T, ct, st)
    vnf = vnt.T
    v_new = jnp.stack([vnf[:, :VD], vnf[:, VD:]], axis=-1)
    return h_new, v_new


# speedup vs baseline: 20.4342x; 1.1849x over previous
import functools

import jax
import jax.numpy as jnp
from jax import lax
from jax.experimental import pallas as pl
from jax.experimental.pallas import tpu as pltpu
from jax.experimental.pallas import tpu_sc as plsc

N = 100000
E = 1600000
SD = 32
VD = 8

NC = 2
NS = 16
NW = NC * NS

F32 = jnp.float32
I32 = jnp.int32

TS_W = 64
TR_W = 48

CH = 80
KG = 5
BG = KG * CH
EPW = E // NW
NB_G = EPW // BG
KS = 10
BS = KS * CH
EPT = E // NS
NB_S = EPT // BS
HALF = N // 2
RPT = HALF // NS

@functools.cache
def _get_gather_edges():
    mesh = plsc.VectorSubcoreMesh(core_axis_name="c", subcore_axis_name="s")

    @functools.partial(
        pl.kernel,
        out_type=(jax.ShapeDtypeStruct((E, TS_W), F32),
                  jax.ShapeDtypeStruct((E, TR_W), F32)),
        mesh=mesh,
        scratch_types=[
            pltpu.VMEM((BG,), I32),
            pltpu.VMEM((BG,), I32),
            pltpu.VMEM((BG, TS_W), F32),
            pltpu.VMEM((BG, TR_W), F32),
            pltpu.SemaphoreType.DMA,
            pltpu.SemaphoreType.DMA,
            pltpu.SemaphoreType.DMA,
        ],
        compiler_params=pltpu.CompilerParams(use_tc_tiling_on_sc=False),
    )
    def _gather_edges(ts_hbm, tr_hbm, snd_hbm, rcv_hbm, xs_hbm, xr_hbm,
                      sidx, ridx, srow, rrow, sem_i, sem_g, sem_w):
        wid = lax.axis_index("s") * NC + lax.axis_index("c")
        base = wid * EPW

        def body(g, carry):
            e0 = base + g * BG
            ci1 = pltpu.async_copy(snd_hbm.at[pl.ds(e0, BG)], sidx, sem_i)
            ci2 = pltpu.async_copy(rcv_hbm.at[pl.ds(e0, BG)], ridx, sem_i)
            ci1.wait()
            ci2.wait()

            @pl.when(g > 0)
            def _():
                p0 = base + (g - 1) * BG
                pltpu.make_async_copy(srow, xs_hbm.at[pl.ds(p0, BG)], sem_w).wait()
                pltpu.make_async_copy(rrow, xr_hbm.at[pl.ds(p0, BG)], sem_w).wait()

            for j in range(KG):
                sl = pl.ds(j * CH, CH)
                pltpu.async_copy(ts_hbm.at[sidx.at[sl]], srow.at[sl], sem_g)
                pltpu.async_copy(tr_hbm.at[ridx.at[sl]], rrow.at[sl], sem_g)
            for j in range(KG):
                sl = pl.ds(j * CH, CH)
                pltpu.make_async_copy(ts_hbm.at[sidx.at[sl]], srow.at[sl], sem_g).wait()
                pltpu.make_async_copy(tr_hbm.at[ridx.at[sl]], rrow.at[sl], sem_g).wait()

            pltpu.async_copy(srow, xs_hbm.at[pl.ds(e0, BG)], sem_w)
            pltpu.async_copy(rrow, xr_hbm.at[pl.ds(e0, BG)], sem_w)
            return carry

        lax.fori_loop(0, NB_G, body, 0)
        pe = base + (NB_G - 1) * BG
        pltpu.make_async_copy(srow, xs_hbm.at[pl.ds(pe, BG)], sem_w).wait()
        pltpu.make_async_copy(rrow, xr_hbm.at[pl.ds(pe, BG)], sem_w).wait()

    return _gather_edges


@functools.cache
def _make_scatter(width):
    mesh = plsc.VectorSubcoreMesh(core_axis_name="c", subcore_axis_name="s")

    @functools.partial(
        pl.kernel,
        out_type=jax.ShapeDtypeStruct((N, width), F32),
        mesh=mesh,
        scratch_types=[
            pltpu.VMEM((BS,), I32),
            pltpu.VMEM((KS, CH), I32),
            pltpu.VMEM((BS, width), F32),
            pltpu.VMEM_SHARED((HALF + 8, width), F32),
            pltpu.SemaphoreType.DMA,
            pltpu.SemaphoreType.DMA,
            pltpu.SemaphoreType.DMA,
        ],
        compiler_params=pltpu.CompilerParams(use_tc_tiling_on_sc=False),
    )
    def _scatter(msg_hbm, rcv_hbm, base_hbm, out_hbm, idx_raw, idx_loc, rows,
                 acc, sem_i, sem_m, sem_sc):
        c = lax.axis_index("c")
        s = lax.axis_index("s")
        nb = c * HALF
        pltpu.sync_copy(base_hbm.at[pl.ds(nb + s * RPT, RPT)],
                        acc.at[pl.ds(s * RPT, RPT)])
        plsc.subcore_barrier()

        def drain_scatters():
            for j in range(KS):
                pltpu.make_async_copy(rows.at[pl.ds(j * CH, CH)],
                                      acc.at[idx_loc.at[j]], sem_sc).wait()

        def body(g, carry):
            e0 = s * EPT + g * BS
            ci = pltpu.async_copy(rcv_hbm.at[pl.ds(e0, BS)], idx_raw, sem_i)

            @pl.when(g > 0)
            def _():
                drain_scatters()

            ci.wait()
            for j in range(KS):
                for k in range(CH // 16):
                    idx = idx_raw[pl.ds(j * CH + k * 16, 16)]
                    loc = idx - nb
                    inb = (loc >= 0) & (loc < HALF)
                    idx_loc[j, pl.ds(k * 16, 16)] = jnp.where(inb, loc, HALF)
            pltpu.async_copy(msg_hbm.at[pl.ds(e0, BS)], rows, sem_m).wait()
            for j in range(KS):
                pltpu.async_copy(rows.at[pl.ds(j * CH, CH)],
                                 acc.at[idx_loc.at[j]], sem_sc, add=True)
            return carry

        lax.fori_loop(0, NB_S, body, 0)
        drain_scatters()
        plsc.subcore_barrier()
        pltpu.sync_copy(acc.at[pl.ds(s * RPT, RPT)],
                        out_hbm.at[pl.ds(nb + s * RPT, RPT)])

    return _scatter


def _silu(x):
    return x * (1.0 / (1.0 + jnp.exp(-x)))


def _prologue_body(vxt_ref, vyt_ref, tht_ref, vrxt_ref, vryt_ref, nrmt_ref,
                   ct_ref, st_ref):
    th = tht_ref[...]
    c = jnp.cos(th)
    s = jnp.sin(th)
    ct_ref[...] = c
    st_ref[...] = s
    vx = vxt_ref[...]
    vy = vyt_ref[...]
    vrxt_ref[...] = vx * c - vy * s
    vryt_ref[...] = vx * s + vy * c
    nrmt_ref[...] = jnp.sqrt(vx * vx + vy * vy)


_prologue = pl.pallas_call(
    _prologue_body,
    out_shape=[
        jax.ShapeDtypeStruct((VD, N), F32),
        jax.ShapeDtypeStruct((VD, N), F32),
        jax.ShapeDtypeStruct((VD, N), F32),
        jax.ShapeDtypeStruct((1, N), F32),
        jax.ShapeDtypeStruct((1, N), F32),
    ],
)


def _epilogue_body(acc_ref, ct_ref, st_ref, out_ref):
    ax = acc_ref[0:VD, :]
    ay = acc_ref[VD:2 * VD, :]
    c = ct_ref[...]
    s = st_ref[...]
    nx = ax * c + ay * s
    ny = ay * c - ax * s
    out_ref[0:VD, :] = nx
    out_ref[VD:2 * VD, :] = ny


_epilogue = pl.pallas_call(
    _epilogue_body,
    out_shape=jax.ShapeDtypeStruct((2 * VD, N), F32),
)


BT = 4000
GT = E // BT


def _edge_compute(xs, xr, w1, b1, w2, b2, w3, b3):
    nrows = xs.shape[0]
    h_s = xs[:, 0:SD]
    vrx = xs[:, SD:SD + VD]
    vry = xs[:, SD + VD:SD + 2 * VD]
    nrm = xs[:, 48:56]
    pxs = xs[:, 56:57]
    pys = xs[:, 57:58]
    ci = xs[:, 58:59]
    si = xs[:, 59:60]
    h_r = xr[:, 0:SD]
    pxr = xr[:, 32:33]
    pyr = xr[:, 33:34]
    cj = xr[:, 34:35]
    sj = xr[:, 35:36]

    dx = pxs - pxr
    dy = pys - pyr
    rr = jnp.sqrt(dx * dx + dy * dy)
    inv = 1.0 / (rr + 1e-8)
    ux = dx * inv
    uy = dy * inv
    cd = ci * cj + si * sj
    sd = si * cj - ci * sj
    vdot = vrx * ux + vry * uy

    pad = jnp.zeros((nrows, 128 - 83), F32)
    x_in = jnp.concatenate([h_s, h_r, rr, cd, sd, nrm, vdot, pad], axis=1)
    h1 = _silu(jnp.dot(x_in, w1, preferred_element_type=F32) + b1)
    h2 = _silu(jnp.dot(h1, w2, preferred_element_type=F32) + b2)
    o = jnp.dot(h2, w3, preferred_element_type=F32) + b3

    a = o[:, 0:8]
    b = o[:, 8:16]
    cgate = o[:, 16:24]
    dh = o[:, 24:56]
    psi = o[:, 56:64]
    g = b + cgate * psi
    mv = jnp.concatenate([a * vrx + g * ux, a * vry + g * uy], axis=1)
    return dh, mv


def _tc_body(xs_ref, xr_ref, w1_ref, b1_ref, w2_ref, b2_ref, w3_ref, b3_ref,
             mh_ref, mv_ref):
    mh, mv = _edge_compute(xs_ref[...], xr_ref[...], w1_ref[...], b1_ref[...],
                           w2_ref[...], b2_ref[...], w3_ref[...], b3_ref[...])
    mh_ref[...] = mh
    mv_ref[...] = mv


_tc_call = pl.pallas_call(
    _tc_body,
    grid=(GT,),
    in_specs=[
        pl.BlockSpec((BT, TS_W), lambda i: (i, 0)),
        pl.BlockSpec((BT, TR_W), lambda i: (i, 0)),
        pl.BlockSpec((128, 192), lambda i: (0, 0)),
        pl.BlockSpec((1, 192), lambda i: (0, 0)),
        pl.BlockSpec((192, 192), lambda i: (0, 0)),
        pl.BlockSpec((1, 192), lambda i: (0, 0)),
        pl.BlockSpec((192, 64), lambda i: (0, 0)),
        pl.BlockSpec((1, 64), lambda i: (0, 0)),
    ],
    out_specs=[
        pl.BlockSpec((BT, SD), lambda i: (i, 0)),
        pl.BlockSpec((BT, 2 * VD), lambda i: (i, 0)),
    ],
    out_shape=[
        jax.ShapeDtypeStruct((E, SD), F32),
        jax.ShapeDtypeStruct((E, 2 * VD), F32),
    ],
)


def _pack_weights(gW1, gb1, gW2, gb2, gW3, gb3, pW1, pb1, pW2, pb2, pW3, pb3,
                  sW1, sb1, sW2, sb2, sW3, sb3):
    f = lambda x: x.astype(F32)
    w1c = jnp.zeros((128, 192), F32)
    w1c = w1c.at[0:67, 0:64].set(f(gW1))
    w1c = w1c.at[0:83, 64:128].set(f(sW1))
    w1c = w1c.at[0:32, 128:192].set(f(pW1))
    b1c = jnp.concatenate([f(gb1), f(sb1), f(pb1)]).reshape(1, 192)
    w2c = jnp.zeros((192, 192), F32)
    w2c = w2c.at[0:64, 0:64].set(f(gW2))
    w2c = w2c.at[64:128, 64:128].set(f(sW2))
    w2c = w2c.at[128:192, 128:192].set(f(pW2))
    b2c = jnp.concatenate([f(gb2), f(sb2), f(pb2)]).reshape(1, 192)
    w3c = jnp.zeros((192, 64), F32)
    w3c = w3c.at[0:64, 0:24].set(f(gW3))
    w3c = w3c.at[64:128, 24:56].set(f(sW3))
    w3c = w3c.at[128:192, 56:64].set(f(pW3))
    b3c = jnp.concatenate([f(gb3), f(sb3), f(pb3)]).reshape(1, 64)
    return w1c, b1c, w2c, b2c, w3c, b3c


def kernel(h, v, midpoint_pos, midpoint_theta, senders, receivers,
           gW1, gb1, gW2, gb2, gW3, gb3,
           pW1, pb1, pW2, pb2, pW3, pb3,
           sW1, sb1, sW2, sb2, sW3, sb3):
    h = h.astype(F32)
    v = v.astype(F32)
    pos = midpoint_pos.astype(F32)
    th = midpoint_theta.astype(F32)
    snd = senders.astype(I32)
    rcv = receivers.astype(I32)

    vxt = v[:, :, 0].T
    vyt = v[:, :, 1].T
    vrxt, vryt, nrmt, ct, st = _prologue(vxt, vyt, th.reshape(1, N))
    vrot_x = vrxt.T
    vrot_y = vryt.T
    nrm = nrmt.T
    c_col = ct.reshape(N, 1)
    s_col = st.reshape(N, 1)

    pad4 = jnp.zeros((N, 4), F32)
    pad12 = jnp.zeros((N, 12), F32)
    ts = jnp.concatenate([h, vrot_x, vrot_y, nrm, pos, c_col, s_col, pad4],
                         axis=1)
    tr = jnp.concatenate([h, pos, c_col, s_col, pad12], axis=1)

    xs, xr = _get_gather_edges()(ts, tr, snd, rcv)

    packed = _pack_weights(gW1, gb1, gW2, gb2, gW3, gb3,
                           pW1, pb1, pW2, pb2, pW3, pb3,
                           sW1, sb1, sW2, sb2, sW3, sb3)
    msg_h, msg_v = _tc_call(xs, xr, *packed)

    h_new = _make_scatter(SD)(msg_h, rcv, h)
    vrot_flat = jnp.concatenate([vrot_x, vrot_y], axis=1)
    vacc = _make_scatter(2 * VD)(msg_v, rcv, vrot_flat)
    vnt = _epilogue(vacc.T, ct, st)
    vnf = vnt.T
    v_new = jnp.stack([vnf[:, :VD], vnf[:, VD:]], axis=-1)
    return h_new, v_new


# speedup vs baseline: 21.9759x; 1.0754x over previous
import functools

import jax
import jax.numpy as jnp
from jax import lax
from jax.experimental import pallas as pl
from jax.experimental.pallas import tpu as pltpu
from jax.experimental.pallas import tpu_sc as plsc

N = 100000
E = 1600000
SD = 32
VD = 8

NC = 2
NS = 16
NW = NC * NS

F32 = jnp.float32
I32 = jnp.int32

TS_W = 64
TR_W = 48

CH = 80
KG = 5
BG = KG * CH
EPW = E // NW
NB_G = EPW // BG
KS = 10
BS = KS * CH
EPT = E // NS
NB_S = EPT // BS
HALF = N // 2
RPT = HALF // NS

@functools.cache
def _get_gather_edges():
    mesh = plsc.VectorSubcoreMesh(core_axis_name="c", subcore_axis_name="s")

    @functools.partial(
        pl.kernel,
        out_type=(jax.ShapeDtypeStruct((E, TS_W), F32),
                  jax.ShapeDtypeStruct((E, TR_W), F32)),
        mesh=mesh,
        scratch_types=[
            pltpu.VMEM((BG,), I32),
            pltpu.VMEM((BG,), I32),
            pltpu.VMEM((BG, TS_W), F32),
            pltpu.VMEM((BG, TR_W), F32),
            pltpu.SemaphoreType.DMA,
            pltpu.SemaphoreType.DMA,
            pltpu.SemaphoreType.DMA,
        ],
        compiler_params=pltpu.CompilerParams(use_tc_tiling_on_sc=False),
    )
    def _gather_edges(ts_hbm, tr_hbm, snd_hbm, rcv_hbm, xs_hbm, xr_hbm,
                      sidx, ridx, srow, rrow, sem_i, sem_g, sem_w):
        wid = lax.axis_index("s") * NC + lax.axis_index("c")
        base = wid * EPW

        def body(g, carry):
            e0 = base + g * BG
            ci1 = pltpu.async_copy(snd_hbm.at[pl.ds(e0, BG)], sidx, sem_i)
            ci2 = pltpu.async_copy(rcv_hbm.at[pl.ds(e0, BG)], ridx, sem_i)
            ci1.wait()
            ci2.wait()

            @pl.when(g > 0)
            def _():
                p0 = base + (g - 1) * BG
                pltpu.make_async_copy(srow, xs_hbm.at[pl.ds(p0, BG)], sem_w).wait()
                pltpu.make_async_copy(rrow, xr_hbm.at[pl.ds(p0, BG)], sem_w).wait()

            for j in range(KG):
                sl = pl.ds(j * CH, CH)
                pltpu.async_copy(ts_hbm.at[sidx.at[sl]], srow.at[sl], sem_g)
                pltpu.async_copy(tr_hbm.at[ridx.at[sl]], rrow.at[sl], sem_g)
            for j in range(KG):
                sl = pl.ds(j * CH, CH)
                pltpu.make_async_copy(ts_hbm.at[sidx.at[sl]], srow.at[sl], sem_g).wait()
                pltpu.make_async_copy(tr_hbm.at[ridx.at[sl]], rrow.at[sl], sem_g).wait()

            pltpu.async_copy(srow, xs_hbm.at[pl.ds(e0, BG)], sem_w)
            pltpu.async_copy(rrow, xr_hbm.at[pl.ds(e0, BG)], sem_w)
            return carry

        lax.fori_loop(0, NB_G, body, 0)
        pe = base + (NB_G - 1) * BG
        pltpu.make_async_copy(srow, xs_hbm.at[pl.ds(pe, BG)], sem_w).wait()
        pltpu.make_async_copy(rrow, xr_hbm.at[pl.ds(pe, BG)], sem_w).wait()

    return _gather_edges


@functools.cache
def _make_scatter(width):
    mesh = plsc.VectorSubcoreMesh(core_axis_name="c", subcore_axis_name="s")

    @functools.partial(
        pl.kernel,
        out_type=jax.ShapeDtypeStruct((N, width), F32),
        mesh=mesh,
        scratch_types=[
            pltpu.VMEM((BS,), I32),
            pltpu.VMEM((KS, CH), I32),
            pltpu.VMEM((BS, width), F32),
            pltpu.VMEM_SHARED((HALF + 8, width), F32),
            pltpu.SemaphoreType.DMA,
            pltpu.SemaphoreType.DMA,
            pltpu.SemaphoreType.DMA,
        ],
        compiler_params=pltpu.CompilerParams(use_tc_tiling_on_sc=False),
    )
    def _scatter(msg_hbm, rcv_hbm, base_hbm, out_hbm, idx_raw, idx_loc, rows,
                 acc, sem_i, sem_m, sem_sc):
        c = lax.axis_index("c")
        s = lax.axis_index("s")
        nb = c * HALF
        pltpu.sync_copy(base_hbm.at[pl.ds(nb + s * RPT, RPT)],
                        acc.at[pl.ds(s * RPT, RPT)])
        plsc.subcore_barrier()

        def drain_scatters():
            for j in range(KS):
                pltpu.make_async_copy(rows.at[pl.ds(j * CH, CH)],
                                      acc.at[idx_loc.at[j]], sem_sc).wait()

        def body(g, carry):
            e0 = s * EPT + g * BS
            ci = pltpu.async_copy(rcv_hbm.at[pl.ds(e0, BS)], idx_raw, sem_i)

            @pl.when(g > 0)
            def _():
                drain_scatters()

            ci.wait()
            for j in range(KS):
                for k in range(CH // 16):
                    idx = idx_raw[pl.ds(j * CH + k * 16, 16)]
                    loc = idx - nb
                    inb = (loc >= 0) & (loc < HALF)
                    idx_loc[j, pl.ds(k * 16, 16)] = jnp.where(inb, loc, HALF)
            pltpu.async_copy(msg_hbm.at[pl.ds(e0, BS)], rows, sem_m).wait()
            for j in range(KS):
                pltpu.async_copy(rows.at[pl.ds(j * CH, CH)],
                                 acc.at[idx_loc.at[j]], sem_sc, add=True)
            return carry

        lax.fori_loop(0, NB_S, body, 0)
        drain_scatters()
        plsc.subcore_barrier()
        pltpu.sync_copy(acc.at[pl.ds(s * RPT, RPT)],
                        out_hbm.at[pl.ds(nb + s * RPT, RPT)])

    return _scatter


def _silu(x):
    return x * (1.0 / (1.0 + jnp.exp(-x)))


BN = 2000
GN = N // BN

def _cols(c_row, s_row):
    cs = jnp.concatenate([c_row, s_row], axis=0)
    eye2 = jnp.eye(2, dtype=F32)
    return lax.dot_general(cs, eye2, (((0,), (0,)), ((), ())),
                           preferred_element_type=F32)


def _prologue_body(h_ref, vx_ref, vy_ref, pos_ref, tht_ref,
                   ts_ref, tr_ref, vrfl_ref, ccol_ref, scol_ref):
    eye1 = jnp.eye(1, dtype=F32)
    th = lax.dot_general(eye1, tht_ref[...], (((1,), (1,)), ((), ())),
                         preferred_element_type=F32)
    c_row = jnp.cos(th)
    s_row = jnp.sin(th)
    cs_col = _cols(c_row, s_row)
    c_col = cs_col[:, 0:1]
    s_col = cs_col[:, 1:2]
    vx = vx_ref[...]
    vy = vy_ref[...]
    vrx = vx * c_col - vy * s_col
    vry = vx * s_col + vy * c_col
    nrm = jnp.sqrt(vx * vx + vy * vy)
    h = h_ref[...]
    pos = pos_ref[...]
    pad4 = jnp.zeros((BN, 4), F32)
    pad12 = jnp.zeros((BN, 12), F32)
    ts_ref[...] = jnp.concatenate(
        [h, vrx, vry, nrm, pos, c_col, s_col, pad4], axis=1)
    tr_ref[...] = jnp.concatenate([h, pos, c_col, s_col, pad12], axis=1)
    vrfl_ref[...] = jnp.concatenate([vrx, vry], axis=1)
    ccol_ref[...] = c_col
    scol_ref[...] = s_col


_prologue = pl.pallas_call(
    _prologue_body,
    grid=(GN,),
    in_specs=[
        pl.BlockSpec((BN, SD), lambda i: (i, 0)),
        pl.BlockSpec((BN, VD), lambda i: (i, 0)),
        pl.BlockSpec((BN, VD), lambda i: (i, 0)),
        pl.BlockSpec((BN, 2), lambda i: (i, 0)),
        pl.BlockSpec((BN, 1), lambda i: (i, 0)),
    ],
    out_specs=[
        pl.BlockSpec((BN, TS_W), lambda i: (i, 0)),
        pl.BlockSpec((BN, TR_W), lambda i: (i, 0)),
        pl.BlockSpec((BN, 2 * VD), lambda i: (i, 0)),
        pl.BlockSpec((BN, 1), lambda i: (i, 0)),
        pl.BlockSpec((BN, 1), lambda i: (i, 0)),
    ],
    out_shape=[
        jax.ShapeDtypeStruct((N, TS_W), F32),
        jax.ShapeDtypeStruct((N, TR_W), F32),
        jax.ShapeDtypeStruct((N, 2 * VD), F32),
        jax.ShapeDtypeStruct((N, 1), F32),
        jax.ShapeDtypeStruct((N, 1), F32),
    ],
)


def _epilogue_body(acc_ref, ccol_ref, scol_ref, out_ref):
    ax = acc_ref[:, 0:VD]
    ay = acc_ref[:, VD:2 * VD]
    c = ccol_ref[...]
    s = scol_ref[...]
    nx = ax * c + ay * s
    ny = ay * c - ax * s
    out_ref[...] = jnp.concatenate([nx, ny], axis=1)


_epilogue = pl.pallas_call(
    _epilogue_body,
    grid=(GN,),
    in_specs=[
        pl.BlockSpec((BN, 2 * VD), lambda i: (i, 0)),
        pl.BlockSpec((BN, 1), lambda i: (i, 0)),
        pl.BlockSpec((BN, 1), lambda i: (i, 0)),
    ],
    out_specs=pl.BlockSpec((BN, 2 * VD), lambda i: (i, 0)),
    out_shape=jax.ShapeDtypeStruct((N, 2 * VD), F32),
)


BT = 4000
GT = E // BT


def _edge_compute(xs, xr, w1, b1, w2, b2, w3, b3):
    nrows = xs.shape[0]
    h_s = xs[:, 0:SD]
    vrx = xs[:, SD:SD + VD]
    vry = xs[:, SD + VD:SD + 2 * VD]
    nrm = xs[:, 48:56]
    pxs = xs[:, 56:57]
    pys = xs[:, 57:58]
    ci = xs[:, 58:59]
    si = xs[:, 59:60]
    h_r = xr[:, 0:SD]
    pxr = xr[:, 32:33]
    pyr = xr[:, 33:34]
    cj = xr[:, 34:35]
    sj = xr[:, 35:36]

    dx = pxs - pxr
    dy = pys - pyr
    rr = jnp.sqrt(dx * dx + dy * dy)
    inv = 1.0 / (rr + 1e-8)
    ux = dx * inv
    uy = dy * inv
    cd = ci * cj + si * sj
    sd = si * cj - ci * sj
    vdot = vrx * ux + vry * uy

    pad = jnp.zeros((nrows, 128 - 83), F32)
    x_in = jnp.concatenate([h_s, h_r, rr, cd, sd, nrm, vdot, pad], axis=1)
    h1 = _silu(jnp.dot(x_in, w1, preferred_element_type=F32) + b1)
    h2 = _silu(jnp.dot(h1, w2, preferred_element_type=F32) + b2)
    o = jnp.dot(h2, w3, preferred_element_type=F32) + b3

    a = o[:, 0:8]
    b = o[:, 8:16]
    cgate = o[:, 16:24]
    dh = o[:, 24:56]
    psi = o[:, 56:64]
    g = b + cgate * psi
    mv = jnp.concatenate([a * vrx + g * ux, a * vry + g * uy], axis=1)
    return dh, mv


def _tc_body(xs_ref, xr_ref, w1_ref, b1_ref, w2_ref, b2_ref, w3_ref, b3_ref,
             mh_ref, mv_ref):
    mh, mv = _edge_compute(xs_ref[...], xr_ref[...], w1_ref[...], b1_ref[...],
                           w2_ref[...], b2_ref[...], w3_ref[...], b3_ref[...])
    mh_ref[...] = mh
    mv_ref[...] = mv


_tc_call = pl.pallas_call(
    _tc_body,
    grid=(GT,),
    in_specs=[
        pl.BlockSpec((BT, TS_W), lambda i: (i, 0)),
        pl.BlockSpec((BT, TR_W), lambda i: (i, 0)),
        pl.BlockSpec((128, 192), lambda i: (0, 0)),
        pl.BlockSpec((1, 192), lambda i: (0, 0)),
        pl.BlockSpec((192, 192), lambda i: (0, 0)),
        pl.BlockSpec((1, 192), lambda i: (0, 0)),
        pl.BlockSpec((192, 64), lambda i: (0, 0)),
        pl.BlockSpec((1, 64), lambda i: (0, 0)),
    ],
    out_specs=[
        pl.BlockSpec((BT, SD), lambda i: (i, 0)),
        pl.BlockSpec((BT, 2 * VD), lambda i: (i, 0)),
    ],
    out_shape=[
        jax.ShapeDtypeStruct((E, SD), F32),
        jax.ShapeDtypeStruct((E, 2 * VD), F32),
    ],
)


def _pack_weights(gW1, gb1, gW2, gb2, gW3, gb3, pW1, pb1, pW2, pb2, pW3, pb3,
                  sW1, sb1, sW2, sb2, sW3, sb3):
    f = lambda x: x.astype(F32)
    w1c = jnp.zeros((128, 192), F32)
    w1c = w1c.at[0:67, 0:64].set(f(gW1))
    w1c = w1c.at[0:83, 64:128].set(f(sW1))
    w1c = w1c.at[0:32, 128:192].set(f(pW1))
    b1c = jnp.concatenate([f(gb1), f(sb1), f(pb1)]).reshape(1, 192)
    w2c = jnp.zeros((192, 192), F32)
    w2c = w2c.at[0:64, 0:64].set(f(gW2))
    w2c = w2c.at[64:128, 64:128].set(f(sW2))
    w2c = w2c.at[128:192, 128:192].set(f(pW2))
    b2c = jnp.concatenate([f(gb2), f(sb2), f(pb2)]).reshape(1, 192)
    w3c = jnp.zeros((192, 64), F32)
    w3c = w3c.at[0:64, 0:24].set(f(gW3))
    w3c = w3c.at[64:128, 24:56].set(f(sW3))
    w3c = w3c.at[128:192, 56:64].set(f(pW3))
    b3c = jnp.concatenate([f(gb3), f(sb3), f(pb3)]).reshape(1, 64)
    return w1c, b1c, w2c, b2c, w3c, b3c


def kernel(h, v, midpoint_pos, midpoint_theta, senders, receivers,
           gW1, gb1, gW2, gb2, gW3, gb3,
           pW1, pb1, pW2, pb2, pW3, pb3,
           sW1, sb1, sW2, sb2, sW3, sb3):
    h = h.astype(F32)
    v = v.astype(F32)
    pos = midpoint_pos.astype(F32)
    th = midpoint_theta.astype(F32)
    snd = senders.astype(I32)
    rcv = receivers.astype(I32)

    vx = v[:, :, 0]
    vy = v[:, :, 1]
    ts, tr, vrot_flat, c_col, s_col = _prologue(h, vx, vy, pos,
                                                th.reshape(N, 1))

    xs, xr = _get_gather_edges()(ts, tr, snd, rcv)

    packed = _pack_weights(gW1, gb1, gW2, gb2, gW3, gb3,
                           pW1, pb1, pW2, pb2, pW3, pb3,
                           sW1, sb1, sW2, sb2, sW3, sb3)
    msg_h, msg_v = _tc_call(xs, xr, *packed)

    h_new = _make_scatter(SD)(msg_h, rcv, h)
    vacc = _make_scatter(2 * VD)(msg_v, rcv, vrot_flat)
    vnf = _epilogue(vacc, c_col, s_col)
    v_new = jnp.stack([vnf[:, :VD], vnf[:, VD:]], axis=-1)
    return h_new, v_new


# speedup vs baseline: 25.3051x; 1.1515x over previous
import functools

import jax
import jax.numpy as jnp
from jax import lax
from jax.experimental import pallas as pl
from jax.experimental.pallas import tpu as pltpu
from jax.experimental.pallas import tpu_sc as plsc

N = 100000
E = 1600000
SD = 32
VD = 8

NC = 2
NS = 16
NW = NC * NS

F32 = jnp.float32
I32 = jnp.int32

TS_W = 64
TR_W = 48

CH = 80
KG = 5
BG = KG * CH
EPW = E // NW
NB_G = EPW // BG
KS = 10
BS = KS * CH
EPT = E // NS
NB_S = EPT // BS
HALF = N // 2
RPT = HALF // NS

@functools.cache
def _get_gather_edges():
    mesh = plsc.VectorSubcoreMesh(core_axis_name="c", subcore_axis_name="s")

    @functools.partial(
        pl.kernel,
        out_type=(jax.ShapeDtypeStruct((E, TS_W), F32),
                  jax.ShapeDtypeStruct((E, TR_W), F32)),
        mesh=mesh,
        scratch_types=[
            pltpu.VMEM((BG,), I32),
            pltpu.VMEM((BG,), I32),
            pltpu.VMEM((BG, TS_W), F32),
            pltpu.VMEM((BG, TR_W), F32),
            pltpu.SemaphoreType.DMA,
            pltpu.SemaphoreType.DMA,
            pltpu.SemaphoreType.DMA,
        ],
        compiler_params=pltpu.CompilerParams(use_tc_tiling_on_sc=False),
    )
    def _gather_edges(ts_hbm, tr_hbm, snd_hbm, rcv_hbm, xs_hbm, xr_hbm,
                      sidx, ridx, srow, rrow, sem_i, sem_g, sem_w):
        wid = lax.axis_index("s") * NC + lax.axis_index("c")
        base = wid * EPW

        def body(g, carry):
            e0 = base + g * BG
            ci1 = pltpu.async_copy(snd_hbm.at[pl.ds(e0, BG)], sidx, sem_i)
            ci2 = pltpu.async_copy(rcv_hbm.at[pl.ds(e0, BG)], ridx, sem_i)
            ci1.wait()
            ci2.wait()

            @pl.when(g > 0)
            def _():
                p0 = base + (g - 1) * BG
                pltpu.make_async_copy(srow, xs_hbm.at[pl.ds(p0, BG)], sem_w).wait()
                pltpu.make_async_copy(rrow, xr_hbm.at[pl.ds(p0, BG)], sem_w).wait()

            for j in range(KG):
                sl = pl.ds(j * CH, CH)
                pltpu.async_copy(ts_hbm.at[sidx.at[sl]], srow.at[sl], sem_g)
                pltpu.async_copy(tr_hbm.at[ridx.at[sl]], rrow.at[sl], sem_g)
            for j in range(KG):
                sl = pl.ds(j * CH, CH)
                pltpu.make_async_copy(ts_hbm.at[sidx.at[sl]], srow.at[sl], sem_g).wait()
                pltpu.make_async_copy(tr_hbm.at[ridx.at[sl]], rrow.at[sl], sem_g).wait()

            pltpu.async_copy(srow, xs_hbm.at[pl.ds(e0, BG)], sem_w)
            pltpu.async_copy(rrow, xr_hbm.at[pl.ds(e0, BG)], sem_w)
            return carry

        lax.fori_loop(0, NB_G, body, 0)
        pe = base + (NB_G - 1) * BG
        pltpu.make_async_copy(srow, xs_hbm.at[pl.ds(pe, BG)], sem_w).wait()
        pltpu.make_async_copy(rrow, xr_hbm.at[pl.ds(pe, BG)], sem_w).wait()

    return _gather_edges


@functools.cache
def _make_scatter(width):
    mesh = plsc.VectorSubcoreMesh(core_axis_name="c", subcore_axis_name="s")

    @functools.partial(
        pl.kernel,
        out_type=jax.ShapeDtypeStruct((N, width), F32),
        mesh=mesh,
        scratch_types=[
            pltpu.VMEM((BS,), I32),
            pltpu.VMEM((KS, CH), I32),
            pltpu.VMEM((BS, width), F32),
            pltpu.VMEM_SHARED((HALF + 8, width), F32),
            pltpu.SemaphoreType.DMA,
            pltpu.SemaphoreType.DMA,
            pltpu.SemaphoreType.DMA,
        ],
        compiler_params=pltpu.CompilerParams(use_tc_tiling_on_sc=False),
    )
    def _scatter(msg_hbm, rcv_hbm, base_hbm, out_hbm, idx_raw, idx_loc, rows,
                 acc, sem_i, sem_m, sem_sc):
        c = lax.axis_index("c")
        s = lax.axis_index("s")
        nb = c * HALF
        pltpu.sync_copy(base_hbm.at[pl.ds(nb + s * RPT, RPT)],
                        acc.at[pl.ds(s * RPT, RPT)])
        plsc.subcore_barrier()

        def drain_scatters():
            for j in range(KS):
                pltpu.make_async_copy(rows.at[pl.ds(j * CH, CH)],
                                      acc.at[idx_loc.at[j]], sem_sc).wait()

        def body(g, carry):
            e0 = s * EPT + g * BS
            ci = pltpu.async_copy(rcv_hbm.at[pl.ds(e0, BS)], idx_raw, sem_i)

            @pl.when(g > 0)
            def _():
                drain_scatters()

            ci.wait()
            for j in range(KS):
                for k in range(CH // 16):
                    idx = idx_raw[pl.ds(j * CH + k * 16, 16)]
                    loc = idx - nb
                    inb = (loc >= 0) & (loc < HALF)
                    idx_loc[j, pl.ds(k * 16, 16)] = jnp.where(inb, loc, HALF)
            pltpu.async_copy(msg_hbm.at[pl.ds(e0, BS)], rows, sem_m).wait()
            for j in range(KS):
                pltpu.async_copy(rows.at[pl.ds(j * CH, CH)],
                                 acc.at[idx_loc.at[j]], sem_sc, add=True)
            return carry

        lax.fori_loop(0, NB_S, body, 0)
        drain_scatters()
        plsc.subcore_barrier()
        pltpu.sync_copy(acc.at[pl.ds(s * RPT, RPT)],
                        out_hbm.at[pl.ds(nb + s * RPT, RPT)])

    return _scatter


def _silu(x):
    return x * (1.0 / (1.0 + jnp.exp(-x)))


BN = 2000
GN = N // BN

def _cols(c_row, s_row):
    cs = jnp.concatenate([c_row, s_row], axis=0)
    eye2 = jnp.eye(2, dtype=F32)
    return lax.dot_general(cs, eye2, (((0,), (0,)), ((), ())),
                           preferred_element_type=F32)


def _prologue_body(h_ref, vx_ref, vy_ref, pos_ref, tht_ref,
                   ts_ref, tr_ref, vrfl_ref, ccol_ref, scol_ref):
    eye1 = jnp.eye(1, dtype=F32)
    th = lax.dot_general(eye1, tht_ref[...], (((1,), (1,)), ((), ())),
                         preferred_element_type=F32)
    c_row = jnp.cos(th)
    s_row = jnp.sin(th)
    cs_col = _cols(c_row, s_row)
    c_col = cs_col[:, 0:1]
    s_col = cs_col[:, 1:2]
    vx = vx_ref[...]
    vy = vy_ref[...]
    vrx = vx * c_col - vy * s_col
    vry = vx * s_col + vy * c_col
    nrm = jnp.sqrt(vx * vx + vy * vy)
    h = h_ref[...]
    pos = pos_ref[...]
    pad4 = jnp.zeros((BN, 4), F32)
    pad12 = jnp.zeros((BN, 12), F32)
    ts_ref[...] = jnp.concatenate(
        [h, vrx, vry, nrm, pos, c_col, s_col, pad4], axis=1)
    tr_ref[...] = jnp.concatenate([h, pos, c_col, s_col, pad12], axis=1)
    vrfl_ref[...] = jnp.concatenate([vrx, vry], axis=1)
    ccol_ref[...] = c_col
    scol_ref[...] = s_col


_prologue = pl.pallas_call(
    _prologue_body,
    grid=(GN,),
    in_specs=[
        pl.BlockSpec((BN, SD), lambda i: (i, 0)),
        pl.BlockSpec((BN, VD), lambda i: (i, 0)),
        pl.BlockSpec((BN, VD), lambda i: (i, 0)),
        pl.BlockSpec((BN, 2), lambda i: (i, 0)),
        pl.BlockSpec((BN, 1), lambda i: (i, 0)),
    ],
    out_specs=[
        pl.BlockSpec((BN, TS_W), lambda i: (i, 0)),
        pl.BlockSpec((BN, TR_W), lambda i: (i, 0)),
        pl.BlockSpec((BN, 2 * VD), lambda i: (i, 0)),
        pl.BlockSpec((BN, 1), lambda i: (i, 0)),
        pl.BlockSpec((BN, 1), lambda i: (i, 0)),
    ],
    out_shape=[
        jax.ShapeDtypeStruct((N, TS_W), F32),
        jax.ShapeDtypeStruct((N, TR_W), F32),
        jax.ShapeDtypeStruct((N, 2 * VD), F32),
        jax.ShapeDtypeStruct((N, 1), F32),
        jax.ShapeDtypeStruct((N, 1), F32),
    ],
)


def _epilogue_body(acc_ref, ccol_ref, scol_ref, out_ref):
    ax = acc_ref[:, 0:VD]
    ay = acc_ref[:, VD:2 * VD]
    c = ccol_ref[...]
    s = scol_ref[...]
    nx = ax * c + ay * s
    ny = ay * c - ax * s
    out_ref[...] = jnp.concatenate([nx, ny], axis=1)


_epilogue = pl.pallas_call(
    _epilogue_body,
    grid=(GN,),
    in_specs=[
        pl.BlockSpec((BN, 2 * VD), lambda i: (i, 0)),
        pl.BlockSpec((BN, 1), lambda i: (i, 0)),
        pl.BlockSpec((BN, 1), lambda i: (i, 0)),
    ],
    out_specs=pl.BlockSpec((BN, 2 * VD), lambda i: (i, 0)),
    out_shape=jax.ShapeDtypeStruct((N, 2 * VD), F32),
)


BT = 4000
GT = E // BT


def _t_to_rows(x_cols, k):
    return lax.dot_general(jnp.eye(k, dtype=F32), x_cols,
                           (((1,), (1,)), ((), ())),
                           preferred_element_type=F32)


def _t_to_cols(x_rows, k):
    return lax.dot_general(x_rows, jnp.eye(k, dtype=F32),
                           (((0,), (0,)), ((), ())),
                           preferred_element_type=F32)


def _edge_compute(xs, xr, w1, b1, w2, b2, w3, b3):
    nrows = xs.shape[0]
    h_s = xs[:, 0:SD]
    h_r = xr[:, 0:SD]
    nrm = xs[:, 48:56]
    vrx_c = xs[:, SD:SD + VD]
    vry_c = xs[:, SD + VD:SD + 2 * VD]

    g_s = _t_to_rows(xs[:, 56:60], 4)
    g_r = _t_to_rows(xr[:, 32:36], 4)
    v_t = _t_to_rows(xs[:, 32:48], 16)

    dx = g_s[0:1, :] - g_r[0:1, :]
    dy = g_s[1:2, :] - g_r[1:2, :]
    rr = jnp.sqrt(dx * dx + dy * dy)
    inv = 1.0 / (rr + 1e-8)
    ux = dx * inv
    uy = dy * inv
    ci = g_s[2:3, :]
    si = g_s[3:4, :]
    cj = g_r[2:3, :]
    sj = g_r[3:4, :]
    cd = ci * cj + si * sj
    sd = si * cj - ci * sj
    vdot_t = v_t[0:VD, :] * ux + v_t[VD:2 * VD, :] * uy

    rcdsd = _t_to_cols(jnp.concatenate([rr, cd, sd], axis=0), 3)
    vdot = _t_to_cols(vdot_t, VD)
    uxy = _t_to_cols(jnp.concatenate([ux, uy], axis=0), 2)
    ux_c = uxy[:, 0:1]
    uy_c = uxy[:, 1:2]

    pad = jnp.zeros((nrows, 128 - 83), F32)
    x_in = jnp.concatenate([h_s, h_r, rcdsd, nrm, vdot, pad], axis=1)
    h1 = _silu(jnp.dot(x_in, w1, preferred_element_type=F32) + b1)
    h2 = _silu(jnp.dot(h1, w2, preferred_element_type=F32) + b2)
    o = jnp.dot(h2, w3, preferred_element_type=F32) + b3

    a = o[:, 0:8]
    b = o[:, 8:16]
    cgate = o[:, 16:24]
    dh = o[:, 24:56]
    psi = o[:, 56:64]
    g = b + cgate * psi
    mv = jnp.concatenate([a * vrx_c + g * ux_c, a * vry_c + g * uy_c], axis=1)
    return dh, mv


def _tc_body(xs_ref, xr_ref, w1_ref, b1_ref, w2_ref, b2_ref, w3_ref, b3_ref,
             mh_ref, mv_ref):
    mh, mv = _edge_compute(xs_ref[...], xr_ref[...], w1_ref[...], b1_ref[...],
                           w2_ref[...], b2_ref[...], w3_ref[...], b3_ref[...])
    mh_ref[...] = mh
    mv_ref[...] = mv


_tc_call = pl.pallas_call(
    _tc_body,
    grid=(GT,),
    in_specs=[
        pl.BlockSpec((BT, TS_W), lambda i: (i, 0)),
        pl.BlockSpec((BT, TR_W), lambda i: (i, 0)),
        pl.BlockSpec((128, 192), lambda i: (0, 0)),
        pl.BlockSpec((1, 192), lambda i: (0, 0)),
        pl.BlockSpec((192, 192), lambda i: (0, 0)),
        pl.BlockSpec((1, 192), lambda i: (0, 0)),
        pl.BlockSpec((192, 64), lambda i: (0, 0)),
        pl.BlockSpec((1, 64), lambda i: (0, 0)),
    ],
    out_specs=[
        pl.BlockSpec((BT, SD), lambda i: (i, 0)),
        pl.BlockSpec((BT, 2 * VD), lambda i: (i, 0)),
    ],
    out_shape=[
        jax.ShapeDtypeStruct((E, SD), F32),
        jax.ShapeDtypeStruct((E, 2 * VD), F32),
    ],
)


def _pack_weights(gW1, gb1, gW2, gb2, gW3, gb3, pW1, pb1, pW2, pb2, pW3, pb3,
                  sW1, sb1, sW2, sb2, sW3, sb3):
    f = lambda x: x.astype(F32)
    w1c = jnp.zeros((128, 192), F32)
    w1c = w1c.at[0:67, 0:64].set(f(gW1))
    w1c = w1c.at[0:83, 64:128].set(f(sW1))
    w1c = w1c.at[0:32, 128:192].set(f(pW1))
    b1c = jnp.concatenate([f(gb1), f(sb1), f(pb1)]).reshape(1, 192)
    w2c = jnp.zeros((192, 192), F32)
    w2c = w2c.at[0:64, 0:64].set(f(gW2))
    w2c = w2c.at[64:128, 64:128].set(f(sW2))
    w2c = w2c.at[128:192, 128:192].set(f(pW2))
    b2c = jnp.concatenate([f(gb2), f(sb2), f(pb2)]).reshape(1, 192)
    w3c = jnp.zeros((192, 64), F32)
    w3c = w3c.at[0:64, 0:24].set(f(gW3))
    w3c = w3c.at[64:128, 24:56].set(f(sW3))
    w3c = w3c.at[128:192, 56:64].set(f(pW3))
    b3c = jnp.concatenate([f(gb3), f(sb3), f(pb3)]).reshape(1, 64)
    return w1c, b1c, w2c, b2c, w3c, b3c


def kernel(h, v, midpoint_pos, midpoint_theta, senders, receivers,
           gW1, gb1, gW2, gb2, gW3, gb3,
           pW1, pb1, pW2, pb2, pW3, pb3,
           sW1, sb1, sW2, sb2, sW3, sb3):
    h = h.astype(F32)
    v = v.astype(F32)
    pos = midpoint_pos.astype(F32)
    th = midpoint_theta.astype(F32)
    snd = senders.astype(I32)
    rcv = receivers.astype(I32)

    vx = v[:, :, 0]
    vy = v[:, :, 1]
    ts, tr, vrot_flat, c_col, s_col = _prologue(h, vx, vy, pos,
                                                th.reshape(N, 1))

    xs, xr = _get_gather_edges()(ts, tr, snd, rcv)

    packed = _pack_weights(gW1, gb1, gW2, gb2, gW3, gb3,
                           pW1, pb1, pW2, pb2, pW3, pb3,
                           sW1, sb1, sW2, sb2, sW3, sb3)
    msg_h, msg_v = _tc_call(xs, xr, *packed)

    h_new = _make_scatter(SD)(msg_h, rcv, h)
    vacc = _make_scatter(2 * VD)(msg_v, rcv, vrot_flat)
    vnf = _epilogue(vacc, c_col, s_col)
    v_new = jnp.stack([vnf[:, :VD], vnf[:, VD:]], axis=-1)
    return h_new, v_new
